# baseline (speedup 1.0000x reference)
import functools

import jax
import jax.numpy as jnp
import numpy as np
from jax import lax
from jax.experimental import pallas as pl
from jax.experimental.pallas import tpu as pltpu
from jax.experimental.pallas import tpu_sc as plsc

N_NODES = 10000
N_EDGES = 5000
NNZ = 160000
D = 256
HALF = D // 2

NC = 2
NS = 16
NW = NC * NS
CH = 64
NCH_E = 80
P_E = 8
NCH_N = 81
P_N = 3

E_PAD = 5120
N_PAD = 10240

BM = 1000
_BN = float(1.0 / np.sqrt(1.0 + 1e-5))
_PREC = lax.Precision.HIGHEST


def _sig(x):
    return 1.0 / (1.0 + jnp.exp(-x))


def _full(shape):
    return pl.BlockSpec(shape, lambda i: tuple(0 for _ in shape))


def _rows(shape):
    return pl.BlockSpec(shape, lambda i: (i,) + tuple(0 for _ in shape[1:]))


def _parts():
    return [pl.BlockSpec((1, BM, HALF), lambda i: (0, i, 0)),
            pl.BlockSpec((1, BM, HALF), lambda i: (1, i, 0))]


def _lin_msg_body(x_ref, wn, bn, wm, bm, x0_ref, m0a_ref, m0b_ref):
    x0 = jnp.dot(x_ref[...], wn[...], preferred_element_type=jnp.float32,
                 precision=_PREC) + bn[...]
    x0_ref[...] = x0
    m0 = _sig(jnp.dot(x0, wm[...], preferred_element_type=jnp.float32,
                      precision=_PREC) + bm[...])
    m0a_ref[...] = m0[:, :HALF]
    m0b_ref[...] = m0[:, HALF:]


def _lin_msg(x, wn, bn, wm, bm):
    m = x.shape[0]
    return pl.pallas_call(
        _lin_msg_body,
        grid=(m // BM,),
        in_specs=[_rows((BM, D)), _full((D, D)), _full((1, D)),
                  _full((D, D)), _full((1, D))],
        out_specs=[_rows((BM, D)), _rows((BM, HALF)), _rows((BM, HALF))],
        out_shape=[jax.ShapeDtypeStruct((m, D), jnp.float32),
                   jax.ShapeDtypeStruct((m, HALF), jnp.float32),
                   jax.ShapeDtypeStruct((m, HALF), jnp.float32)],
    )(x, wn, bn.reshape(1, D), wm, bm.reshape(1, D))


def _lin_body(x_ref, w, b, o_ref):
    o_ref[...] = jnp.dot(x_ref[...], w[...], preferred_element_type=jnp.float32,
                         precision=_PREC) + b[...]


def _lin(x, w, b):
    m = x.shape[0]
    return pl.pallas_call(
        _lin_body,
        grid=(m // BM,),
        in_specs=[_rows((BM, D)), _full((D, D)), _full((1, D))],
        out_specs=_rows((BM, D)),
        out_shape=jax.ShapeDtypeStruct((m, D), jnp.float32),
    )(x, w, b.reshape(1, D))


def _edge_update_body(x1_ref, pa0, pa1, pb0, pb1, wt, wb, b, m1a_ref, m1b_ref,
                      x1n_ref):
    agg = jnp.concatenate([pa0[0] + pa1[0], pb0[0] + pb1[0]], axis=1)
    m1 = _sig(jnp.dot(x1_ref[...], wt[...], preferred_element_type=jnp.float32,
                      precision=_PREC)
              + jnp.dot(agg, wb[...], preferred_element_type=jnp.float32,
                        precision=_PREC) + b[...])
    m1a_ref[...] = m1[:, :HALF]
    m1b_ref[...] = m1[:, HALF:]
    x1n_ref[...] = _BN * _sig(x1_ref[...] + agg)


def _edge_update(x1, pa, pb, wt, wb, b):
    return pl.pallas_call(
        _edge_update_body,
        grid=(N_EDGES // BM,),
        in_specs=[_rows((BM, D))] + _parts() + _parts()
                 + [_full((D, D)), _full((D, D)), _full((1, D))],
        out_specs=[_rows((BM, HALF)), _rows((BM, HALF)), _rows((BM, D))],
        out_shape=[jax.ShapeDtypeStruct((N_EDGES, HALF), jnp.float32),
                   jax.ShapeDtypeStruct((N_EDGES, HALF), jnp.float32),
                   jax.ShapeDtypeStruct((N_EDGES, D), jnp.float32)],
    )(x1, pa, pa, pb, pb, wt, wb, b.reshape(1, D))


def _node_update_msg_body(x0_ref, qa0, qa1, qb0, qb1, wm, bm, x0n_ref,
                          m0a_ref, m0b_ref):
    agg = jnp.concatenate([qa0[0] + qa1[0], qb0[0] + qb1[0]], axis=1)
    x0n = _BN * _sig(x0_ref[...] + agg)
    x0n_ref[...] = x0n
    m0 = _sig(jnp.dot(x0n, wm[...], preferred_element_type=jnp.float32,
                      precision=_PREC) + bm[...])
    m0a_ref[...] = m0[:, :HALF]
    m0b_ref[...] = m0[:, HALF:]


def _node_update_msg(x0, qa, qb, wm, bm):
    return pl.pallas_call(
        _node_update_msg_body,
        grid=(N_NODES // BM,),
        in_specs=[_rows((BM, D))] + _parts() + _parts()
                 + [_full((D, D)), _full((1, D))],
        out_specs=[_rows((BM, D)), _rows((BM, HALF)), _rows((BM, HALF))],
        out_shape=[jax.ShapeDtypeStruct((N_NODES, D), jnp.float32),
                   jax.ShapeDtypeStruct((N_NODES, HALF), jnp.float32),
                   jax.ShapeDtypeStruct((N_NODES, HALF), jnp.float32)],
    )(x0, qa, qa, qb, qb, wm, bm.reshape(1, D))


def _node_update_body(x0_ref, qa0, qa1, qb0, qb1, x0n_ref):
    agg = jnp.concatenate([qa0[0] + qa1[0], qb0[0] + qb1[0]], axis=1)
    x0n_ref[...] = _BN * _sig(x0_ref[...] + agg)


def _node_update(x0, qa, qb):
    return pl.pallas_call(
        _node_update_body,
        grid=(N_NODES // BM,),
        in_specs=[_rows((BM, D))] + _parts() + _parts(),
        out_specs=_rows((BM, D)),
        out_shape=jax.ShapeDtypeStruct((N_NODES, D), jnp.float32),
    )(x0, qa, qa, qb, qb)


_MESH = dict(core_axis_name="core", subcore_axis_name="subcore")


def _seg_phase(src_hbm, out_hbm, gidx_v, sidx_v, bufs, gsems, ssems, acc,
               c, s, rps, nch, p):
    b0 = bufs[0]

    @pl.loop(0, CH)
    def _(r):
        @pl.loop(0, HALF // 16)
        def _(j):
            b0[r, pl.ds(j * 16, 16)] = jnp.zeros((16,), jnp.float32)

    @pl.loop(0, rps // CH)
    def _(t):
        pltpu.sync_copy(b0, acc.at[pl.ds(s * rps + t * CH, CH)])

    plsc.subcore_barrier()

    for b in range(p):
        pltpu.async_copy(src_hbm.at[gidx_v.at[b]], bufs[b], gsems[b])

    @pl.loop(0, nch, step=p)
    def _(j):
        for b in range(p):
            cur = j + b
            pltpu.make_async_copy(src_hbm.at[gidx_v.at[cur]], bufs[b],
                                  gsems[b]).wait()
            pltpu.async_copy(bufs[b], acc.at[sidx_v.at[cur]], ssems[b],
                             add=True)
        for b in range(p):
            cur = j + b
            nxt = cur + p

            @pl.when(nxt < nch)
            def _():
                pltpu.make_async_copy(bufs[b], acc.at[sidx_v.at[cur]],
                                      ssems[b]).wait()
                pltpu.async_copy(src_hbm.at[gidx_v.at[nxt]], bufs[b], gsems[b])

    for b in range(p):
        pltpu.make_async_copy(bufs[b], acc.at[sidx_v.at[nch - p + b]],
                              ssems[b]).wait()

    plsc.subcore_barrier()
    pltpu.sync_copy(acc.at[pl.ds(s * rps, rps)],
                    out_hbm.at[c, pl.ds(s * rps, rps)])


def _seg_sum(srca, srcb, gidx3, sidx3, rows, nch, p):
    mesh = plsc.VectorSubcoreMesh(**_MESH)

    @functools.partial(
        pl.kernel,
        out_type=(jax.ShapeDtypeStruct((NC, rows, HALF), jnp.float32),
                  jax.ShapeDtypeStruct((NC, rows, HALF), jnp.float32)),
        mesh=mesh,
        scratch_types=[
            pltpu.VMEM((nch, CH), jnp.int32),
            pltpu.VMEM((nch, CH), jnp.int32),
        ] + [pltpu.VMEM((CH, HALF), jnp.float32) for _ in range(p)]
          + [pltpu.SemaphoreType.DMA for _ in range(2 * p)]
          + [pltpu.VMEM_SHARED((rows, HALF), jnp.float32)],
    )
    def k(ma_hbm, mb_hbm, gidx_hbm, sidx_hbm, qa_hbm, qb_hbm, gidx_v, sidx_v,
          *rest):
        bufs = rest[:p]
        gsems = rest[p:2 * p]
        ssems = rest[2 * p:3 * p]
        acc = rest[3 * p]
        c = lax.axis_index("core")
        s = lax.axis_index("subcore")
        wid = c * NS + s
        pltpu.sync_copy(gidx_hbm.at[wid], gidx_v)
        pltpu.sync_copy(sidx_hbm.at[wid], sidx_v)
        _seg_phase(ma_hbm, qa_hbm, gidx_v, sidx_v, bufs, gsems, ssems, acc,
                   c, s, rows // NS, nch, p)
        _seg_phase(mb_hbm, qb_hbm, gidx_v, sidx_v, bufs, gsems, ssems, acc,
                   c, s, rows // NS, nch, p)

    return k(srca, srcb, gidx3, sidx3)


def kernel(x_0, x_1, incidence_node_idx, incidence_edge_idx,
           W_node, b_node, W_edge, b_edge,
           l0_W_msg, l0_b_msg, l0_W_e2n, l0_b_e2n,
           l1_W_msg, l1_b_msg, l1_W_e2n, l1_b_e2n):
    nidx = incidence_node_idx.astype(jnp.int32)
    eidx = incidence_edge_idx.astype(jnp.int32)

    def _pad3(idx, fill, nch):
        kpad = NW * nch * CH
        return jnp.concatenate(
            [idx, jnp.full((kpad - NNZ,), fill, jnp.int32)]
        ).reshape(NW, nch, CH)

    nidx_g = _pad3(nidx, 0, NCH_E)
    eidx_s = _pad3(eidx, N_EDGES, NCH_E)
    eidx_g = _pad3(eidx, 0, NCH_N)
    nidx_s = _pad3(nidx, N_NODES, NCH_N)

    x0, m0a, m0b = _lin_msg(x_0, W_node, b_node, l0_W_msg, l0_b_msg)
    x1 = _lin(x_1, W_edge, b_edge)

    pa, pb = _seg_sum(m0a, m0b, nidx_g, eidx_s, E_PAD, NCH_E, P_E)
    m1a, m1b, x1 = _edge_update(x1, pa, pb, l0_W_e2n[:D], l0_W_e2n[D:],
                                l0_b_e2n)
    qa, qb = _seg_sum(m1a, m1b, eidx_g, nidx_s, N_PAD, NCH_N, P_N)
    x0, m0a, m0b = _node_update_msg(x0, qa, qb, l1_W_msg, l1_b_msg)

    pa, pb = _seg_sum(m0a, m0b, nidx_g, eidx_s, E_PAD, NCH_E, P_E)
    m1a, m1b, x1 = _edge_update(x1, pa, pb, l1_W_e2n[:D], l1_W_e2n[D:],
                                l1_b_e2n)
    qa, qb = _seg_sum(m1a, m1b, eidx_g, nidx_s, N_PAD, NCH_N, P_N)
    x0 = _node_update(x0, qa, qb)

    return x0, x1


# speedup vs baseline: 3.7191x; 3.7191x over previous
import functools

import jax
import jax.numpy as jnp
import numpy as np
from jax import lax
from jax.experimental import pallas as pl
from jax.experimental.pallas import tpu as pltpu
from jax.experimental.pallas import tpu_sc as plsc

N_NODES = 10000
N_EDGES = 5000
NNZ = 160000
D = 256
HALF = D // 2

NC = 2
NS = 16
NW = NC * NS
CH = 64
NCH_E = 80
P_E = 8
NCH_N = 81
P_N = 3

E_PAD = 5120
N_PAD = 10240

BM = 1000
_BN = float(1.0 / np.sqrt(1.0 + 1e-5))
_PREC = lax.Precision.HIGHEST


def _sig(x):
    return 1.0 / (1.0 + jnp.exp(-x))


def _full(shape):
    return pl.BlockSpec(shape, lambda i: tuple(0 for _ in shape))


def _rows(shape):
    return pl.BlockSpec(shape, lambda i: (i,) + tuple(0 for _ in shape[1:]))


def _parts():
    return [pl.BlockSpec((1, BM, HALF), lambda i: (0, i, 0)),
            pl.BlockSpec((1, BM, HALF), lambda i: (1, i, 0))]


def _lin_msg_body(x_ref, wn, bn, wm, bm, x0_ref, m0a_ref, m0b_ref):
    x0 = jnp.dot(x_ref[...], wn[...], preferred_element_type=jnp.float32,
                 precision=_PREC) + bn[...]
    x0_ref[...] = x0
    m0 = _sig(jnp.dot(x0, wm[...], preferred_element_type=jnp.float32,
                      precision=_PREC) + bm[...])
    m0a_ref[...] = m0[:, :HALF]
    m0b_ref[...] = m0[:, HALF:]


def _lin_msg(x, wn, bn, wm, bm):
    m = x.shape[0]
    return pl.pallas_call(
        _lin_msg_body,
        grid=(m // BM,),
        in_specs=[_rows((BM, D)), _full((D, D)), _full((1, D)),
                  _full((D, D)), _full((1, D))],
        out_specs=[_rows((BM, D)), _rows((BM, HALF)), _rows((BM, HALF))],
        out_shape=[jax.ShapeDtypeStruct((m, D), jnp.float32),
                   jax.ShapeDtypeStruct((m, HALF), jnp.float32),
                   jax.ShapeDtypeStruct((m, HALF), jnp.float32)],
    )(x, wn, bn.reshape(1, D), wm, bm.reshape(1, D))


def _lin_body(x_ref, w, b, o_ref):
    o_ref[...] = jnp.dot(x_ref[...], w[...], preferred_element_type=jnp.float32,
                         precision=_PREC) + b[...]


def _lin(x, w, b):
    m = x.shape[0]
    return pl.pallas_call(
        _lin_body,
        grid=(m // BM,),
        in_specs=[_rows((BM, D)), _full((D, D)), _full((1, D))],
        out_specs=_rows((BM, D)),
        out_shape=jax.ShapeDtypeStruct((m, D), jnp.float32),
    )(x, w, b.reshape(1, D))


def _edge_update_body(x1_ref, pa0, pa1, pb0, pb1, wt, wb, b, m1a_ref, m1b_ref,
                      x1n_ref):
    agg = jnp.concatenate([pa0[0] + pa1[0], pb0[0] + pb1[0]], axis=1)
    m1 = _sig(jnp.dot(x1_ref[...], wt[...], preferred_element_type=jnp.float32,
                      precision=_PREC)
              + jnp.dot(agg, wb[...], preferred_element_type=jnp.float32,
                        precision=_PREC) + b[...])
    m1a_ref[...] = m1[:, :HALF]
    m1b_ref[...] = m1[:, HALF:]
    x1n_ref[...] = _BN * _sig(x1_ref[...] + agg)


def _edge_update(x1, pa, pb, wt, wb, b):
    return pl.pallas_call(
        _edge_update_body,
        grid=(N_EDGES // BM,),
        in_specs=[_rows((BM, D))] + _parts() + _parts()
                 + [_full((D, D)), _full((D, D)), _full((1, D))],
        out_specs=[_rows((BM, HALF)), _rows((BM, HALF)), _rows((BM, D))],
        out_shape=[jax.ShapeDtypeStruct((N_EDGES, HALF), jnp.float32),
                   jax.ShapeDtypeStruct((N_EDGES, HALF), jnp.float32),
                   jax.ShapeDtypeStruct((N_EDGES, D), jnp.float32)],
    )(x1, pa, pa, pb, pb, wt, wb, b.reshape(1, D))


def _node_update_msg_body(x0_ref, qa0, qa1, qb0, qb1, wm, bm, x0n_ref,
                          m0a_ref, m0b_ref):
    agg = jnp.concatenate([qa0[0] + qa1[0], qb0[0] + qb1[0]], axis=1)
    x0n = _BN * _sig(x0_ref[...] + agg)
    x0n_ref[...] = x0n
    m0 = _sig(jnp.dot(x0n, wm[...], preferred_element_type=jnp.float32,
                      precision=_PREC) + bm[...])
    m0a_ref[...] = m0[:, :HALF]
    m0b_ref[...] = m0[:, HALF:]


def _node_update_msg(x0, qa, qb, wm, bm):
    return pl.pallas_call(
        _node_update_msg_body,
        grid=(N_NODES // BM,),
        in_specs=[_rows((BM, D))] + _parts() + _parts()
                 + [_full((D, D)), _full((1, D))],
        out_specs=[_rows((BM, D)), _rows((BM, HALF)), _rows((BM, HALF))],
        out_shape=[jax.ShapeDtypeStruct((N_NODES, D), jnp.float32),
                   jax.ShapeDtypeStruct((N_NODES, HALF), jnp.float32),
                   jax.ShapeDtypeStruct((N_NODES, HALF), jnp.float32)],
    )(x0, qa, qa, qb, qb, wm, bm.reshape(1, D))


def _node_update_body(x0_ref, qa0, qa1, qb0, qb1, x0n_ref):
    agg = jnp.concatenate([qa0[0] + qa1[0], qb0[0] + qb1[0]], axis=1)
    x0n_ref[...] = _BN * _sig(x0_ref[...] + agg)


def _node_update(x0, qa, qb):
    return pl.pallas_call(
        _node_update_body,
        grid=(N_NODES // BM,),
        in_specs=[_rows((BM, D))] + _parts() + _parts(),
        out_specs=_rows((BM, D)),
        out_shape=jax.ShapeDtypeStruct((N_NODES, D), jnp.float32),
    )(x0, qa, qa, qb, qb)


_MESH = dict(core_axis_name="core", subcore_axis_name="subcore")


def _seg_phase(src_hbm, out_hbm, gidx_v, sidx_v, bufs, gsems, ssems, acc,
               c, s, rps, nch, p):
    b0 = bufs[0]

    @pl.loop(0, CH)
    def _(r):
        @pl.loop(0, HALF // 16)
        def _(j):
            b0[r, pl.ds(j * 16, 16)] = jnp.zeros((16,), jnp.float32)

    @pl.loop(0, rps // CH)
    def _(t):
        pltpu.sync_copy(b0, acc.at[pl.ds(s * rps + t * CH, CH)])

    plsc.subcore_barrier()

    for b in range(p):
        pltpu.async_copy(src_hbm.at[gidx_v.at[b]], bufs[b], gsems[b])

    @pl.loop(0, nch, step=p)
    def _(j):
        for b in range(p):
            cur = j + b
            pltpu.make_async_copy(src_hbm.at[gidx_v.at[cur]], bufs[b],
                                  gsems[b]).wait()
            pltpu.async_copy(bufs[b], acc.at[sidx_v.at[cur]], ssems[b],
                             add=True)
        for b in range(p):
            cur = j + b
            nxt = cur + p

            @pl.when(nxt < nch)
            def _():
                pltpu.make_async_copy(bufs[b], acc.at[sidx_v.at[cur]],
                                      ssems[b]).wait()
                pltpu.async_copy(src_hbm.at[gidx_v.at[nxt]], bufs[b], gsems[b])

    for b in range(p):
        pltpu.make_async_copy(bufs[b], acc.at[sidx_v.at[nch - p + b]],
                              ssems[b]).wait()

    plsc.subcore_barrier()
    pltpu.sync_copy(acc.at[pl.ds(s * rps, rps)],
                    out_hbm.at[c, pl.ds(s * rps, rps)])


def _seg_sum(srca, srcb, gidx3, sidx3, rows, nch, p):
    mesh = plsc.VectorSubcoreMesh(**_MESH)

    @functools.partial(
        pl.kernel,
        out_type=(jax.ShapeDtypeStruct((NC, rows, HALF), jnp.float32),
                  jax.ShapeDtypeStruct((NC, rows, HALF), jnp.float32)),
        mesh=mesh,
        scratch_types=[
            pltpu.VMEM((nch, CH), jnp.int32),
            pltpu.VMEM((nch, CH), jnp.int32),
        ] + [pltpu.VMEM((CH, HALF), jnp.float32) for _ in range(p)]
          + [pltpu.SemaphoreType.DMA for _ in range(2 * p)]
          + [pltpu.VMEM_SHARED((rows, HALF), jnp.float32)],
    )
    def k(ma_hbm, mb_hbm, gidx_hbm, sidx_hbm, qa_hbm, qb_hbm, gidx_v, sidx_v,
          *rest):
        bufs = rest[:p]
        gsems = rest[p:2 * p]
        ssems = rest[2 * p:3 * p]
        acc = rest[3 * p]
        c = lax.axis_index("core")
        s = lax.axis_index("subcore")
        wid = c * NS + s
        pltpu.sync_copy(gidx_hbm.at[wid], gidx_v)
        pltpu.sync_copy(sidx_hbm.at[wid], sidx_v)
        _seg_phase(ma_hbm, qa_hbm, gidx_v, sidx_v, bufs, gsems, ssems, acc,
                   c, s, rows // NS, nch, p)
        _seg_phase(mb_hbm, qb_hbm, gidx_v, sidx_v, bufs, gsems, ssems, acc,
                   c, s, rows // NS, nch, p)

    return k(srca, srcb, gidx3, sidx3)


def kernel(x_0, x_1, incidence_node_idx, incidence_edge_idx,
           W_node, b_node, W_edge, b_edge,
           l0_W_msg, l0_b_msg, l0_W_e2n, l0_b_e2n,
           l1_W_msg, l1_b_msg, l1_W_e2n, l1_b_e2n):
    nidx = incidence_node_idx.astype(jnp.int32)
    eidx = incidence_edge_idx.astype(jnp.int32)

    def _pad3(idx, base, span, nch):
        kpad = NW * nch * CH
        fill = base + jnp.arange(kpad - NNZ, dtype=jnp.int32) % span
        return jnp.concatenate([idx, fill]).reshape(NW, nch, CH)

    nidx_g = _pad3(nidx, 0, N_NODES, NCH_E)
    eidx_s = _pad3(eidx, N_EDGES, E_PAD - N_EDGES, NCH_E)
    eidx_g = _pad3(eidx, 0, N_EDGES, NCH_N)
    nidx_s = _pad3(nidx, N_NODES, N_PAD - N_NODES, NCH_N)

    x0, m0a, m0b = _lin_msg(x_0, W_node, b_node, l0_W_msg, l0_b_msg)
    x1 = _lin(x_1, W_edge, b_edge)

    pa, pb = _seg_sum(m0a, m0b, nidx_g, eidx_s, E_PAD, NCH_E, P_E)
    m1a, m1b, x1 = _edge_update(x1, pa, pb, l0_W_e2n[:D], l0_W_e2n[D:],
                                l0_b_e2n)
    qa, qb = _seg_sum(m1a, m1b, eidx_g, nidx_s, N_PAD, NCH_N, P_N)
    x0, m0a, m0b = _node_update_msg(x0, qa, qb, l1_W_msg, l1_b_msg)

    pa, pb = _seg_sum(m0a, m0b, nidx_g, eidx_s, E_PAD, NCH_E, P_E)
    m1a, m1b, x1 = _edge_update(x1, pa, pb, l1_W_e2n[:D], l1_W_e2n[D:],
                                l1_b_e2n)
    qa, qb = _seg_sum(m1a, m1b, eidx_g, nidx_s, N_PAD, NCH_N, P_N)
    x0 = _node_update(x0, qa, qb)

    return x0, x1


# speedup vs baseline: 3.9145x; 1.0525x over previous
import functools

import jax
import jax.numpy as jnp
import numpy as np
from jax import lax
from jax.experimental import pallas as pl
from jax.experimental.pallas import tpu as pltpu
from jax.experimental.pallas import tpu_sc as plsc

N_NODES = 10000
N_EDGES = 5000
NNZ = 160000
D = 256
HALF = D // 2

NC = 2
NS = 16
NW = NC * NS
CH = 64
NCH_E = 80
P_E = 8
NCH_N = 81
P_N = 3

E_PAD = 5120
N_PAD = 10240

BM = 1000
_BN = float(1.0 / np.sqrt(1.0 + 1e-5))
_PREC = lax.Precision.DEFAULT


def _sig(x):
    return 1.0 / (1.0 + jnp.exp(-x))


def _full(shape):
    return pl.BlockSpec(shape, lambda i: tuple(0 for _ in shape))


def _rows(shape):
    return pl.BlockSpec(shape, lambda i: (i,) + tuple(0 for _ in shape[1:]))


def _parts():
    return [pl.BlockSpec((1, BM, HALF), lambda i: (0, i, 0)),
            pl.BlockSpec((1, BM, HALF), lambda i: (1, i, 0))]


def _lin_msg_body(x_ref, wn, bn, wm, bm, x0_ref, m0a_ref, m0b_ref):
    x0 = jnp.dot(x_ref[...], wn[...], preferred_element_type=jnp.float32,
                 precision=_PREC) + bn[...]
    x0_ref[...] = x0
    m0 = _sig(jnp.dot(x0, wm[...], preferred_element_type=jnp.float32,
                      precision=_PREC) + bm[...])
    m0a_ref[...] = m0[:, :HALF]
    m0b_ref[...] = m0[:, HALF:]


def _lin_msg(x, wn, bn, wm, bm):
    m = x.shape[0]
    return pl.pallas_call(
        _lin_msg_body,
        grid=(m // BM,),
        in_specs=[_rows((BM, D)), _full((D, D)), _full((1, D)),
                  _full((D, D)), _full((1, D))],
        out_specs=[_rows((BM, D)), _rows((BM, HALF)), _rows((BM, HALF))],
        out_shape=[jax.ShapeDtypeStruct((m, D), jnp.float32),
                   jax.ShapeDtypeStruct((m, HALF), jnp.float32),
                   jax.ShapeDtypeStruct((m, HALF), jnp.float32)],
    )(x, wn, bn.reshape(1, D), wm, bm.reshape(1, D))


def _lin_body(x_ref, w, b, o_ref):
    o_ref[...] = jnp.dot(x_ref[...], w[...], preferred_element_type=jnp.float32,
                         precision=_PREC) + b[...]


def _lin(x, w, b):
    m = x.shape[0]
    return pl.pallas_call(
        _lin_body,
        grid=(m // BM,),
        in_specs=[_rows((BM, D)), _full((D, D)), _full((1, D))],
        out_specs=_rows((BM, D)),
        out_shape=jax.ShapeDtypeStruct((m, D), jnp.float32),
    )(x, w, b.reshape(1, D))


def _edge_update_body(x1_ref, pa0, pa1, pb0, pb1, wt, wb, b, m1a_ref, m1b_ref,
                      x1n_ref):
    agg = jnp.concatenate([pa0[0] + pa1[0], pb0[0] + pb1[0]], axis=1)
    m1 = _sig(jnp.dot(x1_ref[...], wt[...], preferred_element_type=jnp.float32,
                      precision=_PREC)
              + jnp.dot(agg, wb[...], preferred_element_type=jnp.float32,
                        precision=_PREC) + b[...])
    m1a_ref[...] = m1[:, :HALF]
    m1b_ref[...] = m1[:, HALF:]
    x1n_ref[...] = _BN * _sig(x1_ref[...] + agg)


def _edge_update(x1, pa, pb, wt, wb, b):
    return pl.pallas_call(
        _edge_update_body,
        grid=(N_EDGES // BM,),
        in_specs=[_rows((BM, D))] + _parts() + _parts()
                 + [_full((D, D)), _full((D, D)), _full((1, D))],
        out_specs=[_rows((BM, HALF)), _rows((BM, HALF)), _rows((BM, D))],
        out_shape=[jax.ShapeDtypeStruct((N_EDGES, HALF), jnp.float32),
                   jax.ShapeDtypeStruct((N_EDGES, HALF), jnp.float32),
                   jax.ShapeDtypeStruct((N_EDGES, D), jnp.float32)],
    )(x1, pa, pa, pb, pb, wt, wb, b.reshape(1, D))


def _node_update_msg_body(x0_ref, qa0, qa1, qb0, qb1, wm, bm, x0n_ref,
                          m0a_ref, m0b_ref):
    agg = jnp.concatenate([qa0[0] + qa1[0], qb0[0] + qb1[0]], axis=1)
    x0n = _BN * _sig(x0_ref[...] + agg)
    x0n_ref[...] = x0n
    m0 = _sig(jnp.dot(x0n, wm[...], preferred_element_type=jnp.float32,
                      precision=_PREC) + bm[...])
    m0a_ref[...] = m0[:, :HALF]
    m0b_ref[...] = m0[:, HALF:]


def _node_update_msg(x0, qa, qb, wm, bm):
    return pl.pallas_call(
        _node_update_msg_body,
        grid=(N_NODES // BM,),
        in_specs=[_rows((BM, D))] + _parts() + _parts()
                 + [_full((D, D)), _full((1, D))],
        out_specs=[_rows((BM, D)), _rows((BM, HALF)), _rows((BM, HALF))],
        out_shape=[jax.ShapeDtypeStruct((N_NODES, D), jnp.float32),
                   jax.ShapeDtypeStruct((N_NODES, HALF), jnp.float32),
                   jax.ShapeDtypeStruct((N_NODES, HALF), jnp.float32)],
    )(x0, qa, qa, qb, qb, wm, bm.reshape(1, D))


def _node_update_body(x0_ref, qa0, qa1, qb0, qb1, x0n_ref):
    agg = jnp.concatenate([qa0[0] + qa1[0], qb0[0] + qb1[0]], axis=1)
    x0n_ref[...] = _BN * _sig(x0_ref[...] + agg)


def _node_update(x0, qa, qb):
    return pl.pallas_call(
        _node_update_body,
        grid=(N_NODES // BM,),
        in_specs=[_rows((BM, D))] + _parts() + _parts(),
        out_specs=_rows((BM, D)),
        out_shape=jax.ShapeDtypeStruct((N_NODES, D), jnp.float32),
    )(x0, qa, qa, qb, qb)


_MESH = dict(core_axis_name="core", subcore_axis_name="subcore")


def _seg_phase(src_hbm, out_hbm, gidx_v, sidx_v, bufs, gsems, ssems, acc,
               c, s, rps, nch, p):
    b0 = bufs[0]

    @pl.loop(0, CH)
    def _(r):
        @pl.loop(0, HALF // 16)
        def _(j):
            b0[r, pl.ds(j * 16, 16)] = jnp.zeros((16,), jnp.float32)

    @pl.loop(0, rps // CH)
    def _(t):
        pltpu.sync_copy(b0, acc.at[pl.ds(s * rps + t * CH, CH)])

    plsc.subcore_barrier()

    for b in range(p):
        pltpu.async_copy(src_hbm.at[gidx_v.at[b]], bufs[b], gsems[b])

    @pl.loop(0, nch, step=p)
    def _(j):
        for b in range(p):
            cur = j + b
            pltpu.make_async_copy(src_hbm.at[gidx_v.at[cur]], bufs[b],
                                  gsems[b]).wait()
            pltpu.async_copy(bufs[b], acc.at[sidx_v.at[cur]], ssems[b],
                             add=True)
        for b in range(p):
            cur = j + b
            nxt = cur + p

            @pl.when(nxt < nch)
            def _():
                pltpu.make_async_copy(bufs[b], acc.at[sidx_v.at[cur]],
                                      ssems[b]).wait()
                pltpu.async_copy(src_hbm.at[gidx_v.at[nxt]], bufs[b], gsems[b])

    for b in range(p):
        pltpu.make_async_copy(bufs[b], acc.at[sidx_v.at[nch - p + b]],
                              ssems[b]).wait()

    plsc.subcore_barrier()
    pltpu.sync_copy(acc.at[pl.ds(s * rps, rps)],
                    out_hbm.at[c, pl.ds(s * rps, rps)])


def _seg_sum(srca, srcb, gidx3, sidx3, rows, nch, p):
    mesh = plsc.VectorSubcoreMesh(**_MESH)

    @functools.partial(
        pl.kernel,
        out_type=(jax.ShapeDtypeStruct((NC, rows, HALF), jnp.float32),
                  jax.ShapeDtypeStruct((NC, rows, HALF), jnp.float32)),
        mesh=mesh,
        scratch_types=[
            pltpu.VMEM((nch, CH), jnp.int32),
            pltpu.VMEM((nch, CH), jnp.int32),
        ] + [pltpu.VMEM((CH, HALF), jnp.float32) for _ in range(p)]
          + [pltpu.SemaphoreType.DMA for _ in range(2 * p)]
          + [pltpu.VMEM_SHARED((rows, HALF), jnp.float32)],
    )
    def k(ma_hbm, mb_hbm, gidx_hbm, sidx_hbm, qa_hbm, qb_hbm, gidx_v, sidx_v,
          *rest):
        bufs = rest[:p]
        gsems = rest[p:2 * p]
        ssems = rest[2 * p:3 * p]
        acc = rest[3 * p]
        c = lax.axis_index("core")
        s = lax.axis_index("subcore")
        wid = c * NS + s
        pltpu.sync_copy(gidx_hbm.at[wid], gidx_v)
        pltpu.sync_copy(sidx_hbm.at[wid], sidx_v)
        _seg_phase(ma_hbm, qa_hbm, gidx_v, sidx_v, bufs, gsems, ssems, acc,
                   c, s, rows // NS, nch, p)
        _seg_phase(mb_hbm, qb_hbm, gidx_v, sidx_v, bufs, gsems, ssems, acc,
                   c, s, rows // NS, nch, p)

    return k(srca, srcb, gidx3, sidx3)


def kernel(x_0, x_1, incidence_node_idx, incidence_edge_idx,
           W_node, b_node, W_edge, b_edge,
           l0_W_msg, l0_b_msg, l0_W_e2n, l0_b_e2n,
           l1_W_msg, l1_b_msg, l1_W_e2n, l1_b_e2n):
    nidx = incidence_node_idx.astype(jnp.int32)
    eidx = incidence_edge_idx.astype(jnp.int32)

    def _pad3(idx, base, span, nch):
        kpad = NW * nch * CH
        fill = base + jnp.arange(kpad - NNZ, dtype=jnp.int32) % span
        return jnp.concatenate([idx, fill]).reshape(NW, nch, CH)

    nidx_g = _pad3(nidx, 0, N_NODES, NCH_E)
    eidx_s = _pad3(eidx, N_EDGES, E_PAD - N_EDGES, NCH_E)
    eidx_g = _pad3(eidx, 0, N_EDGES, NCH_N)
    nidx_s = _pad3(nidx, N_NODES, N_PAD - N_NODES, NCH_N)

    x0, m0a, m0b = _lin_msg(x_0, W_node, b_node, l0_W_msg, l0_b_msg)
    x1 = _lin(x_1, W_edge, b_edge)

    pa, pb = _seg_sum(m0a, m0b, nidx_g, eidx_s, E_PAD, NCH_E, P_E)
    m1a, m1b, x1 = _edge_update(x1, pa, pb, l0_W_e2n[:D], l0_W_e2n[D:],
                                l0_b_e2n)
    qa, qb = _seg_sum(m1a, m1b, eidx_g, nidx_s, N_PAD, NCH_N, P_N)
    x0, m0a, m0b = _node_update_msg(x0, qa, qb, l1_W_msg, l1_b_msg)

    pa, pb = _seg_sum(m0a, m0b, nidx_g, eidx_s, E_PAD, NCH_E, P_E)
    m1a, m1b, x1 = _edge_update(x1, pa, pb, l1_W_e2n[:D], l1_W_e2n[D:],
                                l1_b_e2n)
    qa, qb = _seg_sum(m1a, m1b, eidx_g, nidx_s, N_PAD, NCH_N, P_N)
    x0 = _node_update(x0, qa, qb)

    return x0, x1


# speedup vs baseline: 3.9222x; 1.0020x over previous
import functools

import jax
import jax.numpy as jnp
import numpy as np
from jax import lax
from jax.experimental import pallas as pl
from jax.experimental.pallas import tpu as pltpu
from jax.experimental.pallas import tpu_sc as plsc

N_NODES = 10000
N_EDGES = 5000
NNZ = 160000
D = 256
HALF = D // 2

NC = 2
NS = 16
NW = NC * NS
CH = 64
NCH_E = 80
P_E = 8
NCH_N = 81
P_N = 3

E_PAD = 5120
N_PAD = 10240

BM = 1000
_BN = float(1.0 / np.sqrt(1.0 + 1e-5))
_PREC = lax.Precision.DEFAULT


def _sig(x):
    return 1.0 / (1.0 + jnp.exp(-x))


def _full(shape):
    return pl.BlockSpec(shape, lambda i: tuple(0 for _ in shape))


def _rows(shape):
    return pl.BlockSpec(shape, lambda i: (i,) + tuple(0 for _ in shape[1:]))


def _parts():
    return [pl.BlockSpec((1, BM, HALF), lambda i: (0, i, 0)),
            pl.BlockSpec((1, BM, HALF), lambda i: (1, i, 0))]


def _lin_msg_body(x_ref, wn, bn, wm, bm, x0_ref, m0a_ref, m0b_ref):
    x0 = jnp.dot(x_ref[...], wn[...], preferred_element_type=jnp.float32,
                 precision=_PREC) + bn[...]
    x0_ref[...] = x0
    m0 = _sig(jnp.dot(x0, wm[...], preferred_element_type=jnp.float32,
                      precision=_PREC) + bm[...])
    m0a_ref[...] = m0[:, :HALF]
    m0b_ref[...] = m0[:, HALF:]


def _lin_msg(x, wn, bn, wm, bm):
    m = x.shape[0]
    return pl.pallas_call(
        _lin_msg_body,
        grid=(m // BM,),
        in_specs=[_rows((BM, D)), _full((D, D)), _full((1, D)),
                  _full((D, D)), _full((1, D))],
        out_specs=[_rows((BM, D)), _rows((BM, HALF)), _rows((BM, HALF))],
        out_shape=[jax.ShapeDtypeStruct((m, D), jnp.float32),
                   jax.ShapeDtypeStruct((m, HALF), jnp.float32),
                   jax.ShapeDtypeStruct((m, HALF), jnp.float32)],
    )(x, wn, bn.reshape(1, D), wm, bm.reshape(1, D))


def _lin_body(x_ref, w, b, o_ref):
    o_ref[...] = jnp.dot(x_ref[...], w[...], preferred_element_type=jnp.float32,
                         precision=_PREC) + b[...]


def _lin(x, w, b):
    m = x.shape[0]
    return pl.pallas_call(
        _lin_body,
        grid=(m // BM,),
        in_specs=[_rows((BM, D)), _full((D, D)), _full((1, D))],
        out_specs=_rows((BM, D)),
        out_shape=jax.ShapeDtypeStruct((m, D), jnp.float32),
    )(x, w, b.reshape(1, D))


def _edge_update_body(x1_ref, pa0, pa1, pb0, pb1, wt, wb, b, m1a_ref, m1b_ref,
                      x1n_ref):
    agg = jnp.concatenate([pa0[0] + pa1[0], pb0[0] + pb1[0]], axis=1)
    m1 = _sig(jnp.dot(x1_ref[...], wt[...], preferred_element_type=jnp.float32,
                      precision=_PREC)
              + jnp.dot(agg, wb[...], preferred_element_type=jnp.float32,
                        precision=_PREC) + b[...])
    m1a_ref[...] = m1[:, :HALF]
    m1b_ref[...] = m1[:, HALF:]
    x1n_ref[...] = _BN * _sig(x1_ref[...] + agg)


def _edge_update(x1, pa, pb, wt, wb, b):
    return pl.pallas_call(
        _edge_update_body,
        grid=(N_EDGES // BM,),
        in_specs=[_rows((BM, D))] + _parts() + _parts()
                 + [_full((D, D)), _full((D, D)), _full((1, D))],
        out_specs=[_rows((BM, HALF)), _rows((BM, HALF)), _rows((BM, D))],
        out_shape=[jax.ShapeDtypeStruct((N_EDGES, HALF), jnp.float32),
                   jax.ShapeDtypeStruct((N_EDGES, HALF), jnp.float32),
                   jax.ShapeDtypeStruct((N_EDGES, D), jnp.float32)],
    )(x1, pa, pa, pb, pb, wt, wb, b.reshape(1, D))


def _node_update_msg_body(x0_ref, qa0, qa1, qb0, qb1, wm, bm, x0n_ref,
                          m0a_ref, m0b_ref):
    agg = jnp.concatenate([qa0[0] + qa1[0], qb0[0] + qb1[0]], axis=1)
    x0n = _BN * _sig(x0_ref[...] + agg)
    x0n_ref[...] = x0n
    m0 = _sig(jnp.dot(x0n, wm[...], preferred_element_type=jnp.float32,
                      precision=_PREC) + bm[...])
    m0a_ref[...] = m0[:, :HALF]
    m0b_ref[...] = m0[:, HALF:]


def _node_update_msg(x0, qa, qb, wm, bm):
    return pl.pallas_call(
        _node_update_msg_body,
        grid=(N_NODES // BM,),
        in_specs=[_rows((BM, D))] + _parts() + _parts()
                 + [_full((D, D)), _full((1, D))],
        out_specs=[_rows((BM, D)), _rows((BM, HALF)), _rows((BM, HALF))],
        out_shape=[jax.ShapeDtypeStruct((N_NODES, D), jnp.float32),
                   jax.ShapeDtypeStruct((N_NODES, HALF), jnp.float32),
                   jax.ShapeDtypeStruct((N_NODES, HALF), jnp.float32)],
    )(x0, qa, qa, qb, qb, wm, bm.reshape(1, D))


def _node_update_body(x0_ref, qa0, qa1, qb0, qb1, x0n_ref):
    agg = jnp.concatenate([qa0[0] + qa1[0], qb0[0] + qb1[0]], axis=1)
    x0n_ref[...] = _BN * _sig(x0_ref[...] + agg)


def _node_update(x0, qa, qb):
    return pl.pallas_call(
        _node_update_body,
        grid=(N_NODES // BM,),
        in_specs=[_rows((BM, D))] + _parts() + _parts(),
        out_specs=_rows((BM, D)),
        out_shape=jax.ShapeDtypeStruct((N_NODES, D), jnp.float32),
    )(x0, qa, qa, qb, qb)


_MESH = dict(core_axis_name="core", subcore_axis_name="subcore")


def _seg_phase(src_hbm, out_hbm, gidx_v, sidx_v, bufs, gsems, ssems, acc,
               c, s, rps, nch, p):
    b0 = bufs[0]

    @pl.loop(0, CH)
    def _(r):
        @pl.loop(0, HALF // 16)
        def _(j):
            b0[r, pl.ds(j * 16, 16)] = jnp.zeros((16,), jnp.float32)

    zsem = ssems[0]
    for t in range(rps // CH):
        pltpu.async_copy(b0, acc.at[pl.ds(s * rps + t * CH, CH)], zsem)
    for t in range(rps // CH):
        pltpu.make_async_copy(b0, acc.at[pl.ds(s * rps + t * CH, CH)],
                              zsem).wait()

    plsc.subcore_barrier()

    for b in range(p):
        pltpu.async_copy(src_hbm.at[gidx_v.at[b]], bufs[b], gsems[b])

    @pl.loop(0, nch, step=p)
    def _(j):
        for b in range(p):
            cur = j + b
            pltpu.make_async_copy(src_hbm.at[gidx_v.at[cur]], bufs[b],
                                  gsems[b]).wait()
            pltpu.async_copy(bufs[b], acc.at[sidx_v.at[cur]], ssems[b],
                             add=True)
        for b in range(p):
            cur = j + b
            nxt = cur + p

            @pl.when(nxt < nch)
            def _():
                pltpu.make_async_copy(bufs[b], acc.at[sidx_v.at[cur]],
                                      ssems[b]).wait()
                pltpu.async_copy(src_hbm.at[gidx_v.at[nxt]], bufs[b], gsems[b])

    for b in range(p):
        pltpu.make_async_copy(bufs[b], acc.at[sidx_v.at[nch - p + b]],
                              ssems[b]).wait()

    plsc.subcore_barrier()
    pltpu.sync_copy(acc.at[pl.ds(s * rps, rps)],
                    out_hbm.at[c, pl.ds(s * rps, rps)])


def _seg_sum(srca, srcb, gidx3, sidx3, rows, nch, p):
    mesh = plsc.VectorSubcoreMesh(**_MESH)

    @functools.partial(
        pl.kernel,
        out_type=(jax.ShapeDtypeStruct((NC, rows, HALF), jnp.float32),
                  jax.ShapeDtypeStruct((NC, rows, HALF), jnp.float32)),
        mesh=mesh,
        scratch_types=[
            pltpu.VMEM((nch, CH), jnp.int32),
            pltpu.VMEM((nch, CH), jnp.int32),
        ] + [pltpu.VMEM((CH, HALF), jnp.float32) for _ in range(p)]
          + [pltpu.SemaphoreType.DMA for _ in range(2 * p)]
          + [pltpu.VMEM_SHARED((rows, HALF), jnp.float32)],
    )
    def k(ma_hbm, mb_hbm, gidx_hbm, sidx_hbm, qa_hbm, qb_hbm, gidx_v, sidx_v,
          *rest):
        bufs = rest[:p]
        gsems = rest[p:2 * p]
        ssems = rest[2 * p:3 * p]
        acc = rest[3 * p]
        c = lax.axis_index("core")
        s = lax.axis_index("subcore")
        wid = c * NS + s
        pltpu.sync_copy(gidx_hbm.at[wid], gidx_v)
        pltpu.sync_copy(sidx_hbm.at[wid], sidx_v)
        _seg_phase(ma_hbm, qa_hbm, gidx_v, sidx_v, bufs, gsems, ssems, acc,
                   c, s, rows // NS, nch, p)
        _seg_phase(mb_hbm, qb_hbm, gidx_v, sidx_v, bufs, gsems, ssems, acc,
                   c, s, rows // NS, nch, p)

    return k(srca, srcb, gidx3, sidx3)


def kernel(x_0, x_1, incidence_node_idx, incidence_edge_idx,
           W_node, b_node, W_edge, b_edge,
           l0_W_msg, l0_b_msg, l0_W_e2n, l0_b_e2n,
           l1_W_msg, l1_b_msg, l1_W_e2n, l1_b_e2n):
    nidx = incidence_node_idx.astype(jnp.int32)
    eidx = incidence_edge_idx.astype(jnp.int32)

    def _pad3(idx, base, span, nch):
        kpad = NW * nch * CH
        fill = base + jnp.arange(kpad - NNZ, dtype=jnp.int32) % span
        return jnp.concatenate([idx, fill]).reshape(NW, nch, CH)

    nidx_g = _pad3(nidx, 0, N_NODES, NCH_E)
    eidx_s = _pad3(eidx, N_EDGES, E_PAD - N_EDGES, NCH_E)
    eidx_g = _pad3(eidx, 0, N_EDGES, NCH_N)
    nidx_s = _pad3(nidx, N_NODES, N_PAD - N_NODES, NCH_N)

    x0, m0a, m0b = _lin_msg(x_0, W_node, b_node, l0_W_msg, l0_b_msg)
    x1 = _lin(x_1, W_edge, b_edge)

    pa, pb = _seg_sum(m0a, m0b, nidx_g, eidx_s, E_PAD, NCH_E, P_E)
    m1a, m1b, x1 = _edge_update(x1, pa, pb, l0_W_e2n[:D], l0_W_e2n[D:],
                                l0_b_e2n)
    qa, qb = _seg_sum(m1a, m1b, eidx_g, nidx_s, N_PAD, NCH_N, P_N)
    x0, m0a, m0b = _node_update_msg(x0, qa, qb, l1_W_msg, l1_b_msg)

    pa, pb = _seg_sum(m0a, m0b, nidx_g, eidx_s, E_PAD, NCH_E, P_E)
    m1a, m1b, x1 = _edge_update(x1, pa, pb, l1_W_e2n[:D], l1_W_e2n[D:],
                                l1_b_e2n)
    qa, qb = _seg_sum(m1a, m1b, eidx_g, nidx_s, N_PAD, NCH_N, P_N)
    x0 = _node_update(x0, qa, qb)

    return x0, x1
